# Initial kernel scaffold; baseline (speedup 1.0000x reference)
import jax
import jax.numpy as jnp
from jax.experimental import pallas as pl


def kernel(H_op, H_m, E_seq, E_op2m, W_op, b_op, W_m, b_m):
    raise NotImplementedError("write your pallas kernel here")


# speedup vs baseline: 1.8009x; 1.8009x over previous
import functools

import jax
import jax.numpy as jnp
from jax import lax
from jax.experimental import pallas as pl
from jax.experimental.pallas import tpu as pltpu
from jax.experimental.pallas import tpu_sc as plsc

NC = 2
NS = 16
C = 128
CW = 128
V = 16


def _project(h, w, b, blk):
    n, d = h.shape

    def body(h_ref, w_ref, b_ref, o_ref):
        o_ref[...] = (
            jnp.dot(h_ref[...], w_ref[...], preferred_element_type=jnp.float32)
            + b_ref[...]
        )

    return pl.pallas_call(
        body,
        grid=(n // blk,),
        in_specs=[
            pl.BlockSpec((blk, d), lambda i: (i, 0)),
            pl.BlockSpec((d, d), lambda i: (0, 0)),
            pl.BlockSpec((1, d), lambda i: (0, 0)),
        ],
        out_specs=pl.BlockSpec((blk, d), lambda i: (i, 0)),
        out_shape=jax.ShapeDtypeStruct((n, d), jnp.float32),
    )(h, w, b.reshape(1, d))


def _sc_aggregate(hop_proj, hm_proj, src_idx, dst_idx, z_acc, z_cnt,
                  ones_host, nh, nh_acc, k_chunks, d):
    rps = nh_acc // NS
    mesh = plsc.VectorSubcoreMesh(core_axis_name="c", subcore_axis_name="s")

    @functools.partial(
        pl.kernel,
        out_type=[
            jax.ShapeDtypeStruct((3, NC, nh_acc, d), jnp.float32),
            jax.ShapeDtypeStruct((3, NC, nh_acc, CW), jnp.float32),
        ],
        mesh=mesh,
        scratch_types=[
            pltpu.VMEM_SHARED((nh_acc, d), jnp.float32),
            pltpu.VMEM_SHARED((nh_acc, CW), jnp.float32),
            pltpu.VMEM((C,), jnp.int32),
            pltpu.VMEM((C,), jnp.int32),
            pltpu.VMEM((C, d), jnp.float32),
            pltpu.VMEM((C, CW), jnp.float32),
            pltpu.SemaphoreType.DMA,
        ],
    )
    def agg(hop_hbm, hm_hbm, src_hbm, dst_hbm,
            zacc_hbm, zcnt_hbm, ones_hbm, sums_hbm, cnts_hbm,
            acc, cnt, srcv, dstv, rows, ones, sem):
        cid = lax.axis_index("c")
        sid = lax.axis_index("s")
        base_row = cid * nh
        row_slc = pl.ds(sid * rps, rps)
        pltpu.sync_copy(ones_hbm, ones)
        for a in range(3):
            pltpu.sync_copy(zacc_hbm.at[row_slc], acc.at[row_slc])
            pltpu.sync_copy(zcnt_hbm.at[row_slc], cnt.at[row_slc])
            plsc.subcore_barrier()
            base = (a * NS + sid) * (k_chunks * C)
            table = hop_hbm if a < 2 else hm_hbm

            def body(j, carry):
                off = base + j * C
                pltpu.sync_copy(src_hbm.at[pl.ds(off, C)], srcv)
                pltpu.sync_copy(dst_hbm.at[pl.ds(off, C)], dstv)
                for g in range(C // V):
                    u = dstv[pl.ds(g * V, V)] - base_row
                    keep = jnp.logical_and(u >= 0, u < nh)
                    dstv[pl.ds(g * V, V)] = jnp.where(keep, u, nh)
                pltpu.async_copy(table.at[srcv], rows, sem).wait()
                pltpu.sync_copy(rows, acc.at[dstv], add=True)
                pltpu.sync_copy(ones, cnt.at[dstv], add=True)
                return carry

            lax.fori_loop(0, k_chunks, body, 0)
            plsc.subcore_barrier()
            pltpu.sync_copy(acc.at[row_slc], sums_hbm.at[a, cid, row_slc])
            pltpu.sync_copy(cnt.at[row_slc], cnts_hbm.at[a, cid, row_slc])

    return agg(hop_proj, hm_proj, src_idx, dst_idx, z_acc, z_cnt, ones_host)


def _combine(proj_op, proj_m, sums, cnts, nh, blk):
    n, d = proj_op.shape
    nb_core = nh // blk

    def body(po_ref, pm_ref, s_ref, c_ref, oop_ref, om_ref):
        s = s_ref[...]
        c = c_ref[...]

        def mean(a):
            deg = jnp.maximum(c[a, 0, :, :1], 1.0)
            return s[a, 0] / deg

        oop_ref[...] = jnp.maximum(po_ref[...] + mean(0) + mean(2), 0.0)
        om_ref[...] = jnp.maximum(pm_ref[...] + mean(1), 0.0)

    return pl.pallas_call(
        body,
        grid=(n // blk,),
        in_specs=[
            pl.BlockSpec((blk, d), lambda i: (i, 0)),
            pl.BlockSpec((blk, d), lambda i: (i, 0)),
            pl.BlockSpec((3, 1, blk, d),
                         lambda i: (0, i // nb_core, i % nb_core, 0)),
            pl.BlockSpec((3, 1, blk, CW),
                         lambda i: (0, i // nb_core, i % nb_core, 0)),
        ],
        out_specs=[
            pl.BlockSpec((blk, d), lambda i: (i, 0)),
            pl.BlockSpec((blk, d), lambda i: (i, 0)),
        ],
        out_shape=[
            jax.ShapeDtypeStruct((n, d), jnp.float32),
            jax.ShapeDtypeStruct((n, d), jnp.float32),
        ],
    )(proj_op, proj_m, sums, cnts)


def kernel(H_op, H_m, E_seq, E_op2m, W_op, b_op, W_m, b_m):
    n_op, d = H_op.shape
    n_m = H_m.shape[0]
    e = E_seq.shape[1]
    assert n_op == n_m, "accumulator sizing assumes equal node counts"
    n = n_op
    nh = n // NC

    eps = -(-e // NS)
    k_chunks = -(-eps // C)
    e_pad = NS * k_chunks * C
    nh_acc = -(-(nh + 1) // (NS * 8)) * (NS * 8)

    hop_proj = _project(H_op, W_op, b_op, blk=400)
    hm_proj = _project(H_m, W_m, b_m, blk=400)

    def prep(src, dst):
        src = src.astype(jnp.int32)
        dst = dst.astype(jnp.int32)
        pad = e_pad - e
        src = jnp.concatenate([src, jnp.zeros((pad,), jnp.int32)])
        dst = jnp.concatenate([dst, jnp.full((pad,), n, jnp.int32)])
        return src, dst

    s0, d0 = prep(E_seq[0], E_seq[1])
    s1, d1 = prep(E_op2m[0], E_op2m[1])
    s2, d2 = prep(E_op2m[1], E_op2m[0])
    src_idx = jnp.concatenate([s0, s1, s2])
    dst_idx = jnp.concatenate([d0, d1, d2])

    z_acc = jnp.zeros((nh_acc, d), jnp.float32)
    z_cnt = jnp.zeros((nh_acc, CW), jnp.float32)
    ones_host = jnp.ones((C, CW), jnp.float32)

    sums, cnts = _sc_aggregate(hop_proj, hm_proj, src_idx, dst_idx,
                               z_acc, z_cnt, ones_host, nh, nh_acc,
                               k_chunks, d)
    return _combine(hop_proj, hm_proj, sums, cnts, nh, blk=1000)


# speedup vs baseline: 1.8024x; 1.0009x over previous
import functools

import jax
import jax.numpy as jnp
from jax import lax
from jax.experimental import pallas as pl
from jax.experimental.pallas import tpu as pltpu
from jax.experimental.pallas import tpu_sc as plsc

NC = 2
NS = 16
C = 128
CW = 128
V = 16


def _project(h, w, b, blk):
    n, d = h.shape

    def body(h_ref, w_ref, b_ref, o_ref):
        o_ref[...] = (
            jnp.dot(h_ref[...], w_ref[...], preferred_element_type=jnp.float32)
            + b_ref[...]
        )

    return pl.pallas_call(
        body,
        grid=(n // blk,),
        in_specs=[
            pl.BlockSpec((blk, d), lambda i: (i, 0)),
            pl.BlockSpec((d, d), lambda i: (0, 0)),
            pl.BlockSpec((1, d), lambda i: (0, 0)),
        ],
        out_specs=pl.BlockSpec((blk, d), lambda i: (i, 0)),
        out_shape=jax.ShapeDtypeStruct((n, d), jnp.float32),
    )(h, w, b.reshape(1, d))


def _sc_aggregate(hop_proj, hm_proj, srcE, dstE, z_acc, z_cnt,
                  ones_host, nh, nh_acc, kB, d):
    rps = nh_acc // NS
    mesh = plsc.VectorSubcoreMesh(core_axis_name="c", subcore_axis_name="s")

    @functools.partial(
        pl.kernel,
        out_type=[
            jax.ShapeDtypeStruct((3, NC, nh_acc, d), jnp.float32),
            jax.ShapeDtypeStruct((3, NC, nh_acc, CW), jnp.float32),
        ],
        mesh=mesh,
        scratch_types=[
            pltpu.VMEM_SHARED((nh_acc, d), jnp.float32),
            pltpu.VMEM_SHARED((nh_acc, CW), jnp.float32),
            pltpu.VMEM((C,), jnp.int32),
            pltpu.VMEM((C,), jnp.int32),
            pltpu.VMEM((C, d), jnp.float32),
            pltpu.VMEM((C, CW), jnp.float32),
            pltpu.SemaphoreType.DMA,
        ],
    )
    def agg(hop_hbm, hm_hbm, srcE_hbm, dstE_hbm,
            zacc_hbm, zcnt_hbm, ones_hbm, sums_hbm, cnts_hbm,
            acc, cnt, srcv, dstv, rows, ones, sem):
        cid = lax.axis_index("c")
        sid = lax.axis_index("s")
        base_row = cid * nh
        row_slc = pl.ds(sid * rps, rps)
        pltpu.sync_copy(ones_hbm, ones)
        for a in range(3):
            pltpu.sync_copy(zacc_hbm.at[row_slc], acc.at[row_slc])
            pltpu.sync_copy(zcnt_hbm.at[row_slc], cnt.at[row_slc])
            plsc.subcore_barrier()
            table = hop_hbm if a < 2 else hm_hbm
            base = (a * NS + sid) * (kB * C)

            def body(j, carry):
                off = base + j * C
                pltpu.sync_copy(srcE_hbm.at[pl.ds(off, C)], srcv)
                pltpu.sync_copy(dstE_hbm.at[pl.ds(off, C)], dstv)
                for g in range(C // V):
                    u = dstv[pl.ds(g * V, V)] - base_row
                    keep = jnp.logical_and(u >= 0, u < nh)
                    dstv[pl.ds(g * V, V)] = jnp.where(keep, u, nh)
                pltpu.async_copy(table.at[srcv], rows, sem).wait()
                pltpu.sync_copy(rows, acc.at[dstv], add=True)
                pltpu.sync_copy(ones, cnt.at[dstv], add=True)
                return carry

            lax.fori_loop(0, kB, body, 0)
            plsc.subcore_barrier()
            pltpu.sync_copy(acc.at[row_slc], sums_hbm.at[a, cid, row_slc])
            pltpu.sync_copy(cnt.at[row_slc], cnts_hbm.at[a, cid, row_slc])

    return agg(hop_proj, hm_proj, srcE, dstE, z_acc, z_cnt, ones_host)


def _combine(proj_op, proj_m, sums, cnts, nh, blk):
    n, d = proj_op.shape
    nb_core = nh // blk

    def body(po_ref, pm_ref, s_ref, c_ref, oop_ref, om_ref):
        s = s_ref[...]
        c = c_ref[...]

        def mean(a):
            deg = jnp.maximum(c[a, 0, :, :1], 1.0)
            return s[a, 0] / deg

        oop_ref[...] = jnp.maximum(po_ref[...] + mean(0) + mean(2), 0.0)
        om_ref[...] = jnp.maximum(pm_ref[...] + mean(1), 0.0)

    return pl.pallas_call(
        body,
        grid=(n // blk,),
        in_specs=[
            pl.BlockSpec((blk, d), lambda i: (i, 0)),
            pl.BlockSpec((blk, d), lambda i: (i, 0)),
            pl.BlockSpec((3, 1, blk, d),
                         lambda i: (0, i // nb_core, i % nb_core, 0)),
            pl.BlockSpec((3, 1, blk, CW),
                         lambda i: (0, i // nb_core, i % nb_core, 0)),
        ],
        out_specs=[
            pl.BlockSpec((blk, d), lambda i: (i, 0)),
            pl.BlockSpec((blk, d), lambda i: (i, 0)),
        ],
        out_shape=[
            jax.ShapeDtypeStruct((n, d), jnp.float32),
            jax.ShapeDtypeStruct((n, d), jnp.float32),
        ],
    )(proj_op, proj_m, sums, cnts)


def kernel(H_op, H_m, E_seq, E_op2m, W_op, b_op, W_m, b_m):
    n_op, d = H_op.shape
    n_m = H_m.shape[0]
    e = E_seq.shape[1]
    assert n_op == n_m, "accumulator sizing assumes equal node counts"
    n = n_op
    nh = n // NC

    kB = -(-(-(-e // NS)) // C)
    e_padB = NS * kB * C
    nh_acc = -(-(nh + 1) // (NS * 8)) * (NS * 8)

    hop_proj = _project(H_op, W_op, b_op, blk=400)
    hm_proj = _project(H_m, W_m, b_m, blk=400)

    def prepE(src, dst):
        src = src.astype(jnp.int32)
        dst = dst.astype(jnp.int32)
        pad = e_padB - e
        src = jnp.concatenate([src, jnp.zeros((pad,), jnp.int32)])
        dst = jnp.concatenate([dst, jnp.full((pad,), n, jnp.int32)])
        return src, dst

    se0, de0 = prepE(E_seq[0], E_seq[1])
    se1, de1 = prepE(E_op2m[0], E_op2m[1])
    se2, de2 = prepE(E_op2m[1], E_op2m[0])
    srcE = jnp.concatenate([se0, se1, se2])
    dstE = jnp.concatenate([de0, de1, de2])

    z_acc = jnp.zeros((nh_acc, d), jnp.float32)
    z_cnt = jnp.zeros((nh_acc, CW), jnp.float32)
    ones_host = jnp.ones((C, CW), jnp.float32)

    sums, cnts = _sc_aggregate(hop_proj, hm_proj, srcE, dstE,
                               z_acc, z_cnt, ones_host, nh, nh_acc, kB, d)
    return _combine(hop_proj, hm_proj, sums, cnts, nh, blk=1000)


# speedup vs baseline: 1.8052x; 1.0015x over previous
import functools

import jax
import jax.numpy as jnp
from jax import lax
from jax.experimental import pallas as pl
from jax.experimental.pallas import tpu as pltpu
from jax.experimental.pallas import tpu_sc as plsc

NC = 2
NS = 16
C = 128
CW = 128
V = 16


def _project(h, w, b, blk):
    n, d = h.shape

    def body(h_ref, w_ref, b_ref, o_ref):
        o_ref[...] = (
            jnp.dot(h_ref[...], w_ref[...], preferred_element_type=jnp.float32)
            + b_ref[...]
        )

    return pl.pallas_call(
        body,
        grid=(n // blk,),
        in_specs=[
            pl.BlockSpec((blk, d), lambda i: (i, 0)),
            pl.BlockSpec((d, d), lambda i: (0, 0)),
            pl.BlockSpec((1, d), lambda i: (0, 0)),
        ],
        out_specs=pl.BlockSpec((blk, d), lambda i: (i, 0)),
        out_shape=jax.ShapeDtypeStruct((n, d), jnp.float32),
    )(h, w, b.reshape(1, d))


def _sc_aggregate(hop_proj, hm_proj, srcE, dstE, z_acc, z_cnt,
                  ones_host, nh, nh_acc, kB, d):
    rps = nh_acc // NS
    mesh = plsc.VectorSubcoreMesh(core_axis_name="c", subcore_axis_name="s")

    @functools.partial(
        pl.kernel,
        out_type=[
            jax.ShapeDtypeStruct((3, NC, nh_acc, d), jnp.float32),
            jax.ShapeDtypeStruct((3, NC, nh_acc, CW), jnp.float32),
        ],
        mesh=mesh,
        scratch_types=[
            pltpu.VMEM_SHARED((nh_acc, d), jnp.float32),
            pltpu.VMEM_SHARED((nh_acc, CW), jnp.float32),
            pltpu.VMEM((C,), jnp.int32),
            pltpu.VMEM((C,), jnp.int32),
            pltpu.VMEM((C, d), jnp.float32),
            pltpu.VMEM((C, CW), jnp.float32),
            pltpu.SemaphoreType.DMA,
        ],
    )
    def agg(hop_hbm, hm_hbm, srcE_hbm, dstE_hbm,
            zacc_hbm, zcnt_hbm, ones_hbm, sums_hbm, cnts_hbm,
            acc, cnt, srcv, dstv, rows, ones, sem):
        cid = lax.axis_index("c")
        sid = lax.axis_index("s")
        base_row = cid * nh
        row_slc = pl.ds(sid * rps, rps)
        pltpu.sync_copy(ones_hbm, ones)
        for a in range(3):
            pltpu.sync_copy(zacc_hbm.at[row_slc], acc.at[row_slc])
            pltpu.sync_copy(zcnt_hbm.at[row_slc], cnt.at[row_slc])
            plsc.subcore_barrier()
            table = hop_hbm if a < 2 else hm_hbm
            base = (a * NS + sid) * (kB * C)

            def body(j, carry):
                off = base + j * C
                pltpu.sync_copy(srcE_hbm.at[pl.ds(off, C)], srcv)
                pltpu.sync_copy(dstE_hbm.at[pl.ds(off, C)], dstv)
                gather = pltpu.async_copy(table.at[srcv], rows, sem)
                for g in range(C // V):
                    u = dstv[pl.ds(g * V, V)] - base_row
                    keep = jnp.logical_and(u >= 0, u < nh)
                    dstv[pl.ds(g * V, V)] = jnp.where(keep, u, nh)
                gather.wait()
                pltpu.sync_copy(rows, acc.at[dstv], add=True)
                pltpu.sync_copy(ones, cnt.at[dstv], add=True)
                return carry

            lax.fori_loop(0, kB, body, 0)
            plsc.subcore_barrier()
            pltpu.sync_copy(acc.at[row_slc], sums_hbm.at[a, cid, row_slc])
            pltpu.sync_copy(cnt.at[row_slc], cnts_hbm.at[a, cid, row_slc])

    return agg(hop_proj, hm_proj, srcE, dstE, z_acc, z_cnt, ones_host)


def _combine(proj_op, proj_m, sums, cnts, nh, blk):
    n, d = proj_op.shape
    nb_core = nh // blk

    def body(po_ref, pm_ref, s_ref, c_ref, oop_ref, om_ref):
        s = s_ref[...]
        c = c_ref[...]

        def mean(a):
            deg = jnp.maximum(c[a, 0, :, :1], 1.0)
            return s[a, 0] / deg

        oop_ref[...] = jnp.maximum(po_ref[...] + mean(0) + mean(2), 0.0)
        om_ref[...] = jnp.maximum(pm_ref[...] + mean(1), 0.0)

    return pl.pallas_call(
        body,
        grid=(n // blk,),
        in_specs=[
            pl.BlockSpec((blk, d), lambda i: (i, 0)),
            pl.BlockSpec((blk, d), lambda i: (i, 0)),
            pl.BlockSpec((3, 1, blk, d),
                         lambda i: (0, i // nb_core, i % nb_core, 0)),
            pl.BlockSpec((3, 1, blk, CW),
                         lambda i: (0, i // nb_core, i % nb_core, 0)),
        ],
        out_specs=[
            pl.BlockSpec((blk, d), lambda i: (i, 0)),
            pl.BlockSpec((blk, d), lambda i: (i, 0)),
        ],
        out_shape=[
            jax.ShapeDtypeStruct((n, d), jnp.float32),
            jax.ShapeDtypeStruct((n, d), jnp.float32),
        ],
    )(proj_op, proj_m, sums, cnts)


def kernel(H_op, H_m, E_seq, E_op2m, W_op, b_op, W_m, b_m):
    n_op, d = H_op.shape
    n_m = H_m.shape[0]
    e = E_seq.shape[1]
    assert n_op == n_m, "accumulator sizing assumes equal node counts"
    n = n_op
    nh = n // NC

    kB = -(-(-(-e // NS)) // C)
    e_padB = NS * kB * C
    nh_acc = -(-(nh + 1) // (NS * 8)) * (NS * 8)

    hop_proj = _project(H_op, W_op, b_op, blk=400)
    hm_proj = _project(H_m, W_m, b_m, blk=400)

    def prepE(src, dst):
        src = src.astype(jnp.int32)
        dst = dst.astype(jnp.int32)
        pad = e_padB - e
        src = jnp.concatenate([src, jnp.zeros((pad,), jnp.int32)])
        dst = jnp.concatenate([dst, jnp.full((pad,), n, jnp.int32)])
        return src, dst

    se0, de0 = prepE(E_seq[0], E_seq[1])
    se1, de1 = prepE(E_op2m[0], E_op2m[1])
    se2, de2 = prepE(E_op2m[1], E_op2m[0])
    srcE = jnp.concatenate([se0, se1, se2])
    dstE = jnp.concatenate([de0, de1, de2])

    z_acc = jnp.zeros((nh_acc, d), jnp.float32)
    z_cnt = jnp.zeros((nh_acc, CW), jnp.float32)
    ones_host = jnp.ones((C, CW), jnp.float32)

    sums, cnts = _sc_aggregate(hop_proj, hm_proj, srcE, dstE,
                               z_acc, z_cnt, ones_host, nh, nh_acc, kB, d)
    return _combine(hop_proj, hm_proj, sums, cnts, nh, blk=1000)
